# baseline (speedup 1.0000x reference)
import functools

import jax
import jax.numpy as jnp
from jax import lax
from jax.experimental import pallas as pl
from jax.experimental.pallas import tpu as pltpu
from jax.experimental.pallas import tpu_sc as plsc

_NLVL, _LAT, _LON = 37, 721, 1440
_D = _LAT * _LON
_CH = 3296
_NC = _D // _CH
_NQ = 13
_ITEMS = _NQ * _NC
_L = 16
_NCORES = 2
_NSUB = 16
_NW = _NCORES * _NSUB
_GPW = (_ITEMS + _NW * _L - 1) // (_NW * _L)


def _sc_body(field_hbm, src_hbm, dst_hbm, out_hbm,
             sidx_v, didx_v, buf, gsem, ssem0, ssem1):
    wid = lax.axis_index("s") * _NCORES + lax.axis_index("c")
    pltpu.sync_copy(src_hbm.at[wid], sidx_v)
    pltpu.sync_copy(dst_hbm.at[wid], didx_v)
    ssems = (ssem0, ssem1)
    pending = [None, None]
    for j in range(_GPW):
        b = j % 2
        if pending[b] is not None:
            pending[b].wait()
        pltpu.async_copy(field_hbm.at[sidx_v.at[j]], buf.at[b], gsem).wait()
        h = pltpu.make_async_copy(buf.at[b], out_hbm.at[didx_v.at[j]], ssems[b])
        h.start()
        pending[b] = h
    pending[0].wait()
    pending[1].wait()


_sc_gather = functools.partial(
    pl.kernel,
    out_type=jax.ShapeDtypeStruct((_ITEMS, _CH), jnp.float32),
    mesh=plsc.VectorSubcoreMesh(
        core_axis_name="c", subcore_axis_name="s",
        num_cores=_NCORES, num_subcores=_NSUB),
    scratch_types=[
        pltpu.VMEM((_GPW, _L), jnp.int32),
        pltpu.VMEM((_GPW, _L), jnp.int32),
        pltpu.VMEM((2, _L, _CH), jnp.float32),
        pltpu.SemaphoreType.DMA,
        pltpu.SemaphoreType.DMA,
        pltpu.SemaphoreType.DMA,
    ],
    compiler_params=pltpu.CompilerParams(use_tc_tiling_on_sc=False),
)(_sc_body)


def kernel(field, indices):
    field2 = field.reshape(_NLVL * _NC, _CH)
    t = jnp.minimum(jnp.arange(_NW * _GPW * _L, dtype=jnp.int32), _ITEMS - 1)
    lvl = t // _NC
    chk = t - lvl * _NC
    src = (indices.astype(jnp.int32)[lvl] * _NC + chk).reshape(_NW, _GPW, _L)
    dst = t.reshape(_NW, _GPW, _L)
    out2 = _sc_gather(field2, src, dst)
    return out2.reshape(_NQ, _LAT, _LON)


# speedup vs baseline: 1.0035x; 1.0035x over previous
import functools

import jax
import jax.numpy as jnp
from jax import lax
from jax.experimental import pallas as pl
from jax.experimental.pallas import tpu as pltpu
from jax.experimental.pallas import tpu_sc as plsc

_NLVL, _LAT, _LON = 37, 721, 1440
_D = _LAT * _LON
_CH = 3296
_NC = _D // _CH
_NQ = 13
_ITEMS = _NQ * _NC
_L = 16
_NCORES = 2
_NSUB = 16
_NW = _NCORES * _NSUB
_GPW = (_ITEMS + _NW * _L - 1) // (_NW * _L)


def _sc_body(field_hbm, src_hbm, dst_hbm, out_hbm,
             sidx_v, didx_v, buf, gsem, ssem0, ssem1):
    wid = lax.axis_index("s") * _NCORES + lax.axis_index("c")
    pltpu.sync_copy(src_hbm.at[wid], sidx_v)
    pltpu.sync_copy(dst_hbm.at[wid], didx_v)
    ssems = (ssem0, ssem1)
    pending = [None, None]
    for j in range(_GPW):
        b = j % 2
        if pending[b] is not None:
            pending[b].wait()
        pltpu.async_copy(field_hbm.at[sidx_v.at[j]], buf.at[b], gsem).wait()
        h = pltpu.make_async_copy(buf.at[b], out_hbm.at[didx_v.at[j]], ssems[b])
        h.start()
        pending[b] = h
    pending[0].wait()
    pending[1].wait()


_sc_gather = functools.partial(
    pl.kernel,
    out_type=jax.ShapeDtypeStruct((_ITEMS, _CH), jnp.float32),
    mesh=plsc.VectorSubcoreMesh(
        core_axis_name="c", subcore_axis_name="s",
        num_cores=_NCORES, num_subcores=_NSUB),
    scratch_types=[
        pltpu.VMEM((_GPW, _L), jnp.int32),
        pltpu.VMEM((_GPW, _L), jnp.int32),
        pltpu.VMEM((2, _L, _CH), jnp.float32),
        pltpu.SemaphoreType.DMA,
        pltpu.SemaphoreType.DMA,
        pltpu.SemaphoreType.DMA,
    ],
    compiler_params=pltpu.CompilerParams(use_tc_tiling_on_sc=False),
)(_sc_body)


def kernel(field, indices):
    field2 = field.reshape(_NLVL * _NC, _CH)
    src2d = (indices.astype(jnp.int32)[:, None] * _NC
             + jnp.arange(_NC, dtype=jnp.int32)[None, :])
    src = src2d.reshape(_ITEMS)
    src = jnp.concatenate([src, src[-1:]]).reshape(_NW, _GPW, _L)
    dst = jnp.minimum(jnp.arange(_NW * _GPW * _L, dtype=jnp.int32),
                      _ITEMS - 1).reshape(_NW, _GPW, _L)
    out2 = _sc_gather(field2, src, dst)
    return out2.reshape(_NQ, _LAT, _LON)


# speedup vs baseline: 1.0839x; 1.0801x over previous
import functools

import jax
import jax.numpy as jnp
from jax import lax
from jax.experimental import pallas as pl
from jax.experimental.pallas import tpu as pltpu
from jax.experimental.pallas import tpu_sc as plsc

_NLVL, _LAT, _LON = 37, 721, 1440
_NQ = 13
_NCORES = 2


def kernel(field, indices):
    idx = indices.astype(jnp.int32)
    lvls = [idx[i] for i in range(_NQ)]

    @functools.partial(
        pl.kernel,
        out_type=jax.ShapeDtypeStruct((_NQ, _LAT, _LON), jnp.float32),
        mesh=plsc.ScalarSubcoreMesh(axis_name="c", num_cores=_NCORES),
        scratch_types=[pltpu.SemaphoreType.DMA],
    )
    def run(field_hbm, out_hbm, sem):
        cid = lax.axis_index("c")
        for i in range(_NQ):
            @pl.when(cid == i % _NCORES)
            def _(i=i):
                pltpu.async_copy(field_hbm.at[lvls[i]], out_hbm.at[i], sem)
        for i in range(_NQ):
            @pl.when(cid == i % _NCORES)
            def _(i=i):
                pltpu.make_async_copy(
                    field_hbm.at[lvls[i]], out_hbm.at[i], sem).wait()

    return run(field)


# speedup vs baseline: 7.4556x; 6.8786x over previous
import functools

import jax
import jax.numpy as jnp
from jax import lax
from jax.experimental import pallas as pl
from jax.experimental.pallas import tpu as pltpu
from jax.experimental.pallas import tpu_sc as plsc

_NLVL, _LAT, _LON = 37, 721, 1440
_NQ = 13
_NCORES = 2
_HB = 360


def kernel(field, indices):
    idx = indices.astype(jnp.int32)
    lvls = [idx[i] for i in range(_NQ)]

    @functools.partial(
        pl.kernel,
        out_type=jax.ShapeDtypeStruct((_NQ, _LAT, _LON), jnp.float32),
        mesh=plsc.ScalarSubcoreMesh(axis_name="c", num_cores=_NCORES),
        scratch_types=[
            pltpu.VMEM_SHARED((2, _HB, _LON), jnp.float32),
            pltpu.SemaphoreType.DMA,
            pltpu.SemaphoreType.DMA,
            pltpu.SemaphoreType.DMA,
        ],
    )
    def run(field_hbm, out_hbm, buf, isem, osem0, osem1):
        cid = lax.axis_index("c")
        osems = (osem0, osem1)

        def do_core(my_levels):
            items = [(i, h) for i in my_levels for h in range(2)]
            pending = [None, None]
            for k, (i, h) in enumerate(items):
                b = k % 2
                if pending[b] is not None:
                    pending[b].wait()
                r0 = h * _HB
                pltpu.async_copy(
                    field_hbm.at[lvls[i], pl.ds(r0, _HB)],
                    buf.at[b], isem).wait()
                hh = pltpu.make_async_copy(
                    buf.at[b], out_hbm.at[i, pl.ds(r0, _HB)], osems[b])
                hh.start()
                pending[b] = hh
            for p in pending:
                if p is not None:
                    p.wait()
            for i in my_levels:
                pltpu.async_copy(
                    field_hbm.at[lvls[i], pl.ds(2 * _HB, _LAT - 2 * _HB)],
                    out_hbm.at[i, pl.ds(2 * _HB, _LAT - 2 * _HB)],
                    isem).wait()

        @pl.when(cid == 0)
        def _():
            do_core([i for i in range(_NQ) if i % 2 == 0])

        @pl.when(cid == 1)
        def _():
            do_core([i for i in range(_NQ) if i % 2 == 1])

    return run(field)


# speedup vs baseline: 32.8461x; 4.4055x over previous
import functools

import jax
import jax.numpy as jnp
from jax import lax
from jax.experimental import pallas as pl
from jax.experimental.pallas import tpu as pltpu
from jax.experimental.pallas import tpu_sc as plsc

_NLVL, _LAT, _LON = 37, 721, 1440
_NQ = 13
_NCORES = 2
_HB = _LON // _NCORES


def kernel(field, indices):
    idx = indices.astype(jnp.int32)
    lvls = [idx[i] for i in range(_NQ)]

    @functools.partial(
        pl.kernel,
        out_type=jax.ShapeDtypeStruct((_NQ, _LON, _LAT), jnp.float32),
        mesh=plsc.ScalarSubcoreMesh(axis_name="c", num_cores=_NCORES),
        scratch_types=[
            pltpu.VMEM_SHARED((2, _HB, _LAT), jnp.float32),
            pltpu.SemaphoreType.DMA,
            pltpu.SemaphoreType.DMA,
            pltpu.SemaphoreType.DMA,
        ],
    )
    def run(field_hbm, out_hbm, buf, isem, osem0, osem1):
        cid = lax.axis_index("c")
        r0 = pl.multiple_of(cid * _HB, _HB)
        osems = (osem0, osem1)
        pending = [None, None]
        for i in range(_NQ):
            b = i % 2
            if pending[b] is not None:
                pending[b].wait()
            pltpu.async_copy(
                field_hbm.at[lvls[i], pl.ds(r0, _HB)], buf.at[b], isem).wait()
            h = pltpu.make_async_copy(
                buf.at[b], out_hbm.at[i, pl.ds(r0, _HB)], osems[b])
            h.start()
            pending[b] = h
        for p in pending:
            if p is not None:
                p.wait()

    field_t = jnp.swapaxes(field, 1, 2)
    out_t = run(field_t)
    return jnp.swapaxes(out_t, 1, 2)


# speedup vs baseline: 35.1785x; 1.0710x over previous
import functools

import jax
import jax.numpy as jnp
from jax import lax
from jax.experimental import pallas as pl
from jax.experimental.pallas import tpu as pltpu
from jax.experimental.pallas import tpu_sc as plsc

_NLVL, _LAT, _LON = 37, 721, 1440
_NQ = 13
_NCORES = 2
_HB = _LON // _NCORES
_QB = _HB // 2
_NBUF = 4
_LOOKAHEAD = 2


def kernel(field, indices):
    idx = indices.astype(jnp.int32)
    lvls = [idx[i] for i in range(_NQ)]

    @functools.partial(
        pl.kernel,
        out_type=jax.ShapeDtypeStruct((_NQ, _LON, _LAT), jnp.float32),
        mesh=plsc.ScalarSubcoreMesh(axis_name="c", num_cores=_NCORES),
        scratch_types=[
            pltpu.VMEM_SHARED((_NBUF, _QB, _LAT), jnp.float32),
        ] + [pltpu.SemaphoreType.DMA] * (2 * _NBUF),
    )
    def run(field_hbm, out_hbm, buf, *sems):
        gsems, osems = sems[:_NBUF], sems[_NBUF:]
        cid = lax.axis_index("c")
        r0 = pl.multiple_of(cid * _HB, _HB)
        items = [(i, h) for i in range(_NQ) for h in range(2)]
        n = len(items)
        gh, sh = {}, {}

        def g_start(k):
            i, h = items[k]
            b = k % _NBUF
            if k - _NBUF in sh:
                sh[k - _NBUF].wait()
            g = pltpu.make_async_copy(
                field_hbm.at[lvls[i], pl.ds(r0 + h * _QB, _QB)],
                buf.at[b], gsems[b])
            g.start()
            gh[k] = g

        for k in range(_LOOKAHEAD):
            g_start(k)
        for k in range(n):
            i, h = items[k]
            b = k % _NBUF
            gh[k].wait()
            s = pltpu.make_async_copy(
                buf.at[b], out_hbm.at[i, pl.ds(r0 + h * _QB, _QB)], osems[b])
            s.start()
            sh[k] = s
            if k + _LOOKAHEAD < n:
                g_start(k + _LOOKAHEAD)
        for k in range(n - _NBUF, n):
            sh[k].wait()

    field_t = jnp.swapaxes(field, 1, 2)
    out_t = run(field_t)
    return jnp.swapaxes(out_t, 1, 2)


# speedup vs baseline: 37.2035x; 1.0576x over previous
import functools

import jax
import jax.numpy as jnp
from jax import lax
from jax.experimental import pallas as pl
from jax.experimental.pallas import tpu as pltpu
from jax.experimental.pallas import tpu_sc as plsc

_NLVL, _LAT, _LON = 37, 721, 1440
_NQ = 13
_NCORES = 2
_HB = _LON // _NCORES
_QB = _HB // 3
_NBUF = 6
_LOOKAHEAD = 3


def kernel(field, indices):
    idx = indices.astype(jnp.int32)
    lvls = [idx[i] for i in range(_NQ)]

    @functools.partial(
        pl.kernel,
        out_type=jax.ShapeDtypeStruct((_NQ, _LON, _LAT), jnp.float32),
        mesh=plsc.ScalarSubcoreMesh(axis_name="c", num_cores=_NCORES),
        scratch_types=[
            pltpu.VMEM_SHARED((_NBUF, _QB, _LAT), jnp.float32),
        ] + [pltpu.SemaphoreType.DMA] * (2 * _NBUF),
    )
    def run(field_hbm, out_hbm, buf, *sems):
        gsems, osems = sems[:_NBUF], sems[_NBUF:]
        cid = lax.axis_index("c")
        r0 = pl.multiple_of(cid * _HB, _HB)
        items = [(i, h) for i in range(_NQ) for h in range(3)]
        n = len(items)
        gh, sh = {}, {}

        def g_start(k):
            i, h = items[k]
            b = k % _NBUF
            if k - _NBUF in sh:
                sh[k - _NBUF].wait()
            g = pltpu.make_async_copy(
                field_hbm.at[lvls[i], pl.ds(r0 + h * _QB, _QB)],
                buf.at[b], gsems[b])
            g.start()
            gh[k] = g

        for k in range(_LOOKAHEAD):
            g_start(k)
        for k in range(n):
            i, h = items[k]
            b = k % _NBUF
            gh[k].wait()
            s = pltpu.make_async_copy(
                buf.at[b], out_hbm.at[i, pl.ds(r0 + h * _QB, _QB)], osems[b])
            s.start()
            sh[k] = s
            if k + _LOOKAHEAD < n:
                g_start(k + _LOOKAHEAD)
        for k in range(n - _NBUF, n):
            sh[k].wait()

    field_t = jnp.swapaxes(field, 1, 2)
    out_t = run(field_t)
    return jnp.swapaxes(out_t, 1, 2)
